# Initial kernel scaffold; baseline (speedup 1.0000x reference)
import jax
import jax.numpy as jnp
from jax.experimental import pallas as pl


def kernel(q, k, v, attn_scores_cmp, spatial_size):
    raise NotImplementedError("write your pallas kernel here")


# speedup vs baseline: 1.4463x; 1.4463x over previous
import functools

import jax
import jax.numpy as jnp
from jax import lax
from jax.experimental import pallas as pl
from jax.experimental.pallas import tpu as pltpu
from jax.experimental.pallas import tpu_sc as plsc

_NSEL = 64
_CHUNK = 1024


def _topk_indices(acc):
    n = acc.shape[-1]
    vrow = acc.reshape(1, n)
    vcols = lax.broadcast_in_dim(vrow, (n, n), (0, 1))
    vcol1 = jnp.transpose(vrow, (1, 0))
    vrows = lax.broadcast_in_dim(vcol1, (n, n), (0, 1))
    jj = lax.broadcasted_iota(jnp.int32, (n, n), 0)
    ii = lax.broadcasted_iota(jnp.int32, (n, n), 1)
    beats = (vrows > vcols) | ((vrows == vcols) & (jj < ii))
    rank_row = jnp.sum(beats.astype(jnp.int32), axis=0, keepdims=True)
    rank_col = jnp.transpose(rank_row, (1, 0))
    rank_b = lax.broadcast_in_dim(rank_col, (n, _NSEL), (0, 1))
    rr = lax.broadcasted_iota(jnp.int32, (n, _NSEL), 1)
    ivals = lax.broadcasted_iota(jnp.int32, (n, _NSEL), 0)
    idxmat = jnp.where(rank_b == rr, ivals, 0)
    return jnp.sum(idxmat, axis=0, keepdims=True).reshape(1, 1, _NSEL)


def _reduce_topk(scores):
    B, H, N, NC = scores.shape
    nchunk = N // _CHUNK

    def body(s_ref, acc_ref, idx_ref):
        h = pl.program_id(1)
        c = pl.program_id(2)
        partial = jnp.sum(s_ref[...], axis=2)
        first = (h == 0) & (c == 0)

        @pl.when(first)
        def _():
            acc_ref[...] = partial

        @pl.when(~first)
        def _():
            acc_ref[...] = acc_ref[...] + partial

        @pl.when((h == H - 1) & (c == nchunk - 1))
        def _():
            idx_ref[...] = _topk_indices(acc_ref[...])

    acc, idx = pl.pallas_call(
        body,
        grid=(B, H, nchunk),
        in_specs=[pl.BlockSpec((1, 1, _CHUNK, NC), lambda b, h, c: (b, h, c, 0))],
        out_specs=[
            pl.BlockSpec((1, 1, NC), lambda b, h, c: (b, 0, 0)),
            pl.BlockSpec((1, 1, _NSEL), lambda b, h, c: (b, 0, 0)),
        ],
        out_shape=[
            jax.ShapeDtypeStruct((B, 1, NC), jnp.float32),
            jax.ShapeDtypeStruct((B, 1, _NSEL), jnp.int32),
        ],
    )(scores)
    del acc
    return idx.reshape(B, _NSEL)


def _gather_sc(kr, vr, idx):
    mesh = plsc.VectorSubcoreMesh(core_axis_name="c", subcore_axis_name="s")

    @functools.partial(
        pl.kernel,
        mesh=mesh,
        out_type=[
            jax.ShapeDtypeStruct((1024, 1024), jnp.float32),
            jax.ShapeDtypeStruct((1024, 1024), jnp.float32),
        ],
        scratch_types=[
            pltpu.VMEM((16,), jnp.int32),
            pltpu.VMEM((64,), jnp.int32),
            pltpu.VMEM((64, 1024), jnp.float32),
            pltpu.SemaphoreType.DMA,
        ],
    )
    def sck(kr_h, vr_h, idx_h, gk_h, gv_h, idxv, rows, inb, sem):
        wid = lax.axis_index("s") * 2 + lax.axis_index("c")
        tensor = wid // 16
        w2 = wid % 16
        b = w2 // 4
        s0 = (w2 % 4) * 16

        pltpu.sync_copy(idx_h.at[b, pl.ds(s0, 16)], idxv)
        ivec = idxv[...]
        base = b * 1024 + lax.div(ivec, 16) * 64 + lax.rem(ivec, 16)
        for r in range(4):
            rows[pl.ds(r * 16, 16)] = base + r * 16

        @pl.when(tensor == 0)
        def _():
            pltpu.async_copy(kr_h.at[rows], inb, sem).wait()
            pltpu.sync_copy(inb, gk_h.at[pl.ds(w2 * 64, 64), :])

        @pl.when(tensor == 1)
        def _():
            pltpu.async_copy(vr_h.at[rows], inb, sem).wait()
            pltpu.sync_copy(inb, gv_h.at[pl.ds(w2 * 64, 64), :])

    return sck(kr, vr, idx)


def _scramble_tc(gk, gv):
    gk6 = gk.reshape(16, 4, 16, 4, 256)
    gv6 = gv.reshape(16, 4, 16, 4, 256)

    def body(k_ref, v_ref, ok_ref, ov_ref):
        for tl in range(16):
            xk = k_ref[0, :, tl, :, :].reshape(16, 256)
            ok_ref[tl] = jnp.transpose(xk, (1, 0))
            xv = v_ref[0, :, tl, :, :].reshape(16, 256)
            ov_ref[tl] = jnp.transpose(xv, (1, 0))

    in_spec = pl.BlockSpec((1, 4, 16, 4, 256), lambda w: (w, 0, 0, 0, 0))
    out_spec = pl.BlockSpec((16, 256, 16), lambda w: (w, 0, 0))
    tk, tv = pl.pallas_call(
        body,
        grid=(16,),
        in_specs=[in_spec, in_spec],
        out_specs=[out_spec, out_spec],
        out_shape=[
            jax.ShapeDtypeStruct((256, 256, 16), jnp.float32),
            jax.ShapeDtypeStruct((256, 256, 16), jnp.float32),
        ],
    )(gk6, gv6)
    return tk, tv


def kernel(q, k, v, attn_scores_cmp, spatial_size):
    del q, spatial_size
    B = attn_scores_cmp.shape[0]
    indices = _reduce_topk(attn_scores_cmp)
    kr = k.reshape(B * 1024, 1024)
    vr = v.reshape(B * 1024, 1024)
    gk, gv = _gather_sc(kr, vr, indices)
    tk, tv = _scramble_tc(gk, gv)
    k_slc = tk.reshape(B, _NSEL * 16, 256)
    v_slc = tv.reshape(B, _NSEL * 16, 256)
    return (k_slc, v_slc, indices)


# speedup vs baseline: 1.8503x; 1.2793x over previous
import functools

import jax
import jax.numpy as jnp
from jax import lax
from jax.experimental import pallas as pl
from jax.experimental.pallas import tpu as pltpu
from jax.experimental.pallas import tpu_sc as plsc

_NSEL = 64
_CHUNK = 1024


def _topk_indices(acc):
    n = acc.shape[-1]
    vrow = acc.reshape(1, n)
    vcols = lax.broadcast_in_dim(vrow, (n, n), (0, 1))
    vcol1 = jnp.transpose(vrow, (1, 0))
    vrows = lax.broadcast_in_dim(vcol1, (n, n), (0, 1))
    jj = lax.broadcasted_iota(jnp.int32, (n, n), 0)
    ii = lax.broadcasted_iota(jnp.int32, (n, n), 1)
    beats = (vrows > vcols) | ((vrows == vcols) & (jj < ii))
    rank_row = jnp.sum(beats.astype(jnp.int32), axis=0, keepdims=True)
    rank_col = jnp.transpose(rank_row, (1, 0))
    rank_b = lax.broadcast_in_dim(rank_col, (n, _NSEL), (0, 1))
    rr = lax.broadcasted_iota(jnp.int32, (n, _NSEL), 1)
    ivals = lax.broadcasted_iota(jnp.int32, (n, _NSEL), 0)
    idxmat = jnp.where(rank_b == rr, ivals, 0)
    return jnp.sum(idxmat, axis=0, keepdims=True).reshape(1, 1, _NSEL)


def _reduce_topk(scores):
    B, H, N, NC = scores.shape
    nchunk = N // _CHUNK

    def body(*refs):
        s_refs, (acc_ref, idx_ref) = refs[:H], refs[H:]
        c = pl.program_id(1)
        partial = s_refs[0][...].sum(axis=2)
        for j in range(1, H):
            partial = partial + s_refs[j][...].sum(axis=2)

        @pl.when(c == 0)
        def _():
            acc_ref[...] = partial

        @pl.when(c != 0)
        def _():
            acc_ref[...] = acc_ref[...] + partial

        @pl.when(c == nchunk - 1)
        def _():
            idx_ref[...] = _topk_indices(acc_ref[...])

    def mk_spec(j):
        return pl.BlockSpec((1, 1, _CHUNK, NC), lambda b, c, j=j: (b, j, c, 0))

    acc, idx = pl.pallas_call(
        body,
        grid=(B, nchunk),
        in_specs=[mk_spec(j) for j in range(H)],
        out_specs=[
            pl.BlockSpec((1, 1, NC), lambda b, c: (b, 0, 0)),
            pl.BlockSpec((1, 1, _NSEL), lambda b, c: (b, 0, 0)),
        ],
        out_shape=[
            jax.ShapeDtypeStruct((B, 1, NC), jnp.float32),
            jax.ShapeDtypeStruct((B, 1, _NSEL), jnp.int32),
        ],
    )(*([scores] * H))
    del acc
    return idx.reshape(B, _NSEL)


def _gather_sc(kr, vr, idx):
    mesh = plsc.VectorSubcoreMesh(core_axis_name="c", subcore_axis_name="s")

    @functools.partial(
        pl.kernel,
        mesh=mesh,
        out_type=[
            jax.ShapeDtypeStruct((1024, 1024), jnp.float32),
            jax.ShapeDtypeStruct((1024, 1024), jnp.float32),
        ],
        scratch_types=[
            pltpu.VMEM((16,), jnp.int32),
            pltpu.VMEM((64,), jnp.int32),
            pltpu.VMEM((64, 1024), jnp.float32),
            pltpu.SemaphoreType.DMA,
        ],
    )
    def sck(kr_h, vr_h, idx_h, gk_h, gv_h, idxv, rows, inb, sem):
        wid = lax.axis_index("s") * 2 + lax.axis_index("c")
        tensor = wid // 16
        w2 = wid % 16
        b = w2 // 4
        s0 = (w2 % 4) * 16

        pltpu.sync_copy(idx_h.at[b, pl.ds(s0, 16)], idxv)
        ivec = idxv[...]
        base = b * 1024 + lax.div(ivec, 16) * 64 + lax.rem(ivec, 16)
        for r in range(4):
            rows[pl.ds(r * 16, 16)] = base + r * 16

        @pl.when(tensor == 0)
        def _():
            pltpu.async_copy(kr_h.at[rows], inb, sem).wait()
            pltpu.sync_copy(inb, gk_h.at[pl.ds(w2 * 64, 64), :])

        @pl.when(tensor == 1)
        def _():
            pltpu.async_copy(vr_h.at[rows], inb, sem).wait()
            pltpu.sync_copy(inb, gv_h.at[pl.ds(w2 * 64, 64), :])

    return sck(kr, vr, idx)


def _scramble_tc(gk, gv):
    gk6 = gk.reshape(16, 4, 16, 4, 256)
    gv6 = gv.reshape(16, 4, 16, 4, 256)

    def body(k_ref, v_ref, ok_ref, ov_ref):
        for tl in range(16):
            xk = k_ref[0, :, tl, :, :].reshape(16, 256)
            ok_ref[tl] = jnp.transpose(xk, (1, 0))
            xv = v_ref[0, :, tl, :, :].reshape(16, 256)
            ov_ref[tl] = jnp.transpose(xv, (1, 0))

    in_spec = pl.BlockSpec((1, 4, 16, 4, 256), lambda w: (w, 0, 0, 0, 0))
    out_spec = pl.BlockSpec((16, 256, 16), lambda w: (w, 0, 0))
    tk, tv = pl.pallas_call(
        body,
        grid=(16,),
        in_specs=[in_spec, in_spec],
        out_specs=[out_spec, out_spec],
        out_shape=[
            jax.ShapeDtypeStruct((256, 256, 16), jnp.float32),
            jax.ShapeDtypeStruct((256, 256, 16), jnp.float32),
        ],
    )(gk6, gv6)
    return tk, tv


def kernel(q, k, v, attn_scores_cmp, spatial_size):
    del q, spatial_size
    B = attn_scores_cmp.shape[0]
    indices = _reduce_topk(attn_scores_cmp)
    kr = k.reshape(B * 1024, 1024)
    vr = v.reshape(B * 1024, 1024)
    gk, gv = _gather_sc(kr, vr, indices)
    tk, tv = _scramble_tc(gk, gv)
    k_slc = tk.reshape(B, _NSEL * 16, 256)
    v_slc = tv.reshape(B, _NSEL * 16, 256)
    return (k_slc, v_slc, indices)


# speedup vs baseline: 1.8711x; 1.0112x over previous
import functools

import jax
import jax.numpy as jnp
from jax import lax
from jax.experimental import pallas as pl
from jax.experimental.pallas import tpu as pltpu
from jax.experimental.pallas import tpu_sc as plsc

_NSEL = 64
_CHUNK = 1024


def _topk_indices(acc):
    n = acc.shape[-1]
    vrow = acc.reshape(1, n)
    vcols = lax.broadcast_in_dim(vrow, (n, n), (0, 1))
    vcol1 = jnp.transpose(vrow, (1, 0))
    vrows = lax.broadcast_in_dim(vcol1, (n, n), (0, 1))
    jj = lax.broadcasted_iota(jnp.int32, (n, n), 0)
    ii = lax.broadcasted_iota(jnp.int32, (n, n), 1)
    beats = (vrows > vcols) | ((vrows == vcols) & (jj < ii))
    rank_row = jnp.sum(beats.astype(jnp.int32), axis=0, keepdims=True)
    rank_col = jnp.transpose(rank_row, (1, 0))
    rank_b = lax.broadcast_in_dim(rank_col, (n, _NSEL), (0, 1))
    rr = lax.broadcasted_iota(jnp.int32, (n, _NSEL), 1)
    ivals = lax.broadcasted_iota(jnp.int32, (n, _NSEL), 0)
    idxmat = jnp.where(rank_b == rr, ivals, 0)
    return jnp.sum(idxmat, axis=0, keepdims=True).reshape(1, 1, _NSEL)


def _reduce_topk(scores):
    B, H, N, NC = scores.shape
    nchunk = N // _CHUNK

    nstream = H * 2

    def body(*refs):
        s_refs, (acc_ref, idx_ref) = refs[:nstream], refs[nstream:]
        c = pl.program_id(1)
        partial = s_refs[0][...].sum(axis=2)
        for j in range(1, nstream):
            partial = partial + s_refs[j][...].sum(axis=2)

        @pl.when(c == 0)
        def _():
            acc_ref[...] = partial

        @pl.when(c != 0)
        def _():
            acc_ref[...] = acc_ref[...] + partial

        @pl.when(c == nchunk // 2 - 1)
        def _():
            idx_ref[...] = _topk_indices(acc_ref[...])

    def mk_spec(j):
        h, p = j // 2, j % 2
        return pl.BlockSpec(
            (1, 1, _CHUNK, NC), lambda b, c, h=h, p=p: (b, h, c * 2 + p, 0))

    acc, idx = pl.pallas_call(
        body,
        grid=(B, nchunk // 2),
        in_specs=[mk_spec(j) for j in range(nstream)],
        out_specs=[
            pl.BlockSpec((1, 1, NC), lambda b, c: (b, 0, 0)),
            pl.BlockSpec((1, 1, _NSEL), lambda b, c: (b, 0, 0)),
        ],
        out_shape=[
            jax.ShapeDtypeStruct((B, 1, NC), jnp.float32),
            jax.ShapeDtypeStruct((B, 1, _NSEL), jnp.int32),
        ],
    )(*([scores] * nstream))
    del acc
    return idx.reshape(B, _NSEL)


def _gather_sc(kr, vr, idx):
    mesh = plsc.VectorSubcoreMesh(core_axis_name="c", subcore_axis_name="s")

    @functools.partial(
        pl.kernel,
        mesh=mesh,
        out_type=[
            jax.ShapeDtypeStruct((1024, 1024), jnp.float32),
            jax.ShapeDtypeStruct((1024, 1024), jnp.float32),
        ],
        scratch_types=[
            pltpu.VMEM((16,), jnp.int32),
            pltpu.VMEM((64,), jnp.int32),
            pltpu.VMEM((64, 1024), jnp.float32),
            pltpu.SemaphoreType.DMA,
        ],
    )
    def sck(kr_h, vr_h, idx_h, gk_h, gv_h, idxv, rows, inb, sem):
        wid = lax.axis_index("s") * 2 + lax.axis_index("c")
        tensor = wid // 16
        w2 = wid % 16
        b = w2 // 4
        s0 = (w2 % 4) * 16

        pltpu.sync_copy(idx_h.at[b, pl.ds(s0, 16)], idxv)
        ivec = idxv[...]
        base = b * 1024 + lax.div(ivec, 16) * 64 + lax.rem(ivec, 16)
        for r in range(4):
            rows[pl.ds(r * 16, 16)] = base + r * 16

        @pl.when(tensor == 0)
        def _():
            pltpu.async_copy(kr_h.at[rows], inb, sem).wait()
            pltpu.sync_copy(inb, gk_h.at[pl.ds(w2 * 64, 64), :])

        @pl.when(tensor == 1)
        def _():
            pltpu.async_copy(vr_h.at[rows], inb, sem).wait()
            pltpu.sync_copy(inb, gv_h.at[pl.ds(w2 * 64, 64), :])

    return sck(kr, vr, idx)


def _scramble_tc(gk, gv):
    gk6 = gk.reshape(16, 4, 16, 4, 256)
    gv6 = gv.reshape(16, 4, 16, 4, 256)

    def body(k_ref, v_ref, ok_ref, ov_ref):
        for tl in range(16):
            xk = k_ref[0, :, tl, :, :].reshape(16, 256)
            ok_ref[tl] = jnp.transpose(xk, (1, 0))
            xv = v_ref[0, :, tl, :, :].reshape(16, 256)
            ov_ref[tl] = jnp.transpose(xv, (1, 0))

    in_spec = pl.BlockSpec((1, 4, 16, 4, 256), lambda w: (w, 0, 0, 0, 0))
    out_spec = pl.BlockSpec((16, 256, 16), lambda w: (w, 0, 0))
    tk, tv = pl.pallas_call(
        body,
        grid=(16,),
        in_specs=[in_spec, in_spec],
        out_specs=[out_spec, out_spec],
        out_shape=[
            jax.ShapeDtypeStruct((256, 256, 16), jnp.float32),
            jax.ShapeDtypeStruct((256, 256, 16), jnp.float32),
        ],
    )(gk6, gv6)
    return tk, tv


def kernel(q, k, v, attn_scores_cmp, spatial_size):
    del q, spatial_size
    B = attn_scores_cmp.shape[0]
    indices = _reduce_topk(attn_scores_cmp)
    kr = k.reshape(B * 1024, 1024)
    vr = v.reshape(B * 1024, 1024)
    gk, gv = _gather_sc(kr, vr, indices)
    tk, tv = _scramble_tc(gk, gv)
    k_slc = tk.reshape(B, _NSEL * 16, 256)
    v_slc = tv.reshape(B, _NSEL * 16, 256)
    return (k_slc, v_slc, indices)
